# baseline (speedup 1.0000x reference)
import functools

import jax
import jax.numpy as jnp
from jax import lax
from jax.experimental import pallas as pl
from jax.experimental.pallas import tpu as pltpu
from jax.experimental.pallas import tpu_sc as plsc

_B = 2
_K = 1024
_C = 96
_H = 512
_W = 512
_HW = _H * _W
_N = _B * _HW
_NC = 2
_NS = 16
_NW = _NC * _NS
_RPW = _N // _NW
_G = 128
_GPC = 4
_R = _G * _GPC
_NCHUNK = _RPW // _R


def _build():
    mesh = plsc.VectorSubcoreMesh(core_axis_name="c", subcore_axis_name="s")

    @functools.partial(
        pl.kernel,
        mesh=mesh,
        compiler_params=pltpu.CompilerParams(use_tc_tiling_on_sc=False),
        out_type=jax.ShapeDtypeStruct((_B, _H, _W, _C), jnp.float32),
        scratch_types=[
            pltpu.VMEM((_RPW // _G, _G), jnp.int32),
            pltpu.VMEM((2, _R, _C), jnp.float32),
            pltpu.SemaphoreType.DMA,
            pltpu.SemaphoreType.DMA,
        ],
    )
    def gather_kernel(idx_hbm, table_hbm, out_hbm, idx_v, rows_v, sem0, sem1):
        sems = (sem0, sem1)
        wid = lax.axis_index("s") * _NC + lax.axis_index("c")
        base = wid * _RPW
        off = (base // _HW) * _K

        pltpu.sync_copy(idx_hbm.at[pl.ds(wid * (_RPW // _G), _RPW // _G)], idx_v)

        def add_off(r, carry):
            for i in range(_G // 16):
                sl = pl.ds(i * 16, 16)
                idx_v[r, sl] = idx_v[r, sl] + off
            return carry

        lax.fori_loop(0, _RPW // _G, add_off, 0)

        def fire(g, b):
            for j in range(_GPC):
                pltpu.async_copy(
                    table_hbm.at[idx_v.at[g * _GPC + j]],
                    rows_v.at[b].at[pl.ds(j * _G, _G)],
                    sems[b],
                )

        def drain(b):
            pltpu.make_async_copy(
                table_hbm.at[pl.ds(0, _R)], rows_v.at[b], sems[b]
            ).wait()

        def writeback(g, b):
            row0 = base + g * _R
            pltpu.sync_copy(
                rows_v.at[b],
                out_hbm.at[row0 // _HW, (row0 // _W) % _H],
            )

        fire(0, 0)
        fire(1, 1)

        def body(h, carry):
            g = 2 * h
            more = h + 1 < _NCHUNK // 2
            drain(0)
            writeback(g, 0)

            @pl.when(more)
            def _():
                fire(g + 2, 0)

            drain(1)
            writeback(g + 1, 1)

            @pl.when(more)
            def _():
                fire(g + 3, 1)

            return carry

        lax.fori_loop(0, _NCHUNK // 2, body, 0)

    return gather_kernel


_gather = jax.jit(_build())


def kernel(pooled_feature_map, superpixel_map):
    table = pooled_feature_map.reshape(_B * _K, _C)
    idx = superpixel_map.reshape(_N // _G, _G)
    return _gather(idx, table)


# speedup vs baseline: 1.4476x; 1.4476x over previous
import functools

import jax
import jax.numpy as jnp
from jax import lax
from jax.experimental import pallas as pl
from jax.experimental.pallas import tpu as pltpu
from jax.experimental.pallas import tpu_sc as plsc

_B = 2
_K = 1024
_C = 96
_CP = 128
_H = 512
_W = 512
_HW = _H * _W
_N = _B * _HW
_NC = 2
_NS = 16
_NW = _NC * _NS
_RPW = _N // _NW
_G = 128
_GPC = 2
_R = _G * _GPC
_NCHUNK = _RPW // _R


def _build():
    mesh = plsc.VectorSubcoreMesh(core_axis_name="c", subcore_axis_name="s")

    @functools.partial(
        pl.kernel,
        mesh=mesh,
        compiler_params=pltpu.CompilerParams(use_tc_tiling_on_sc=True),
        out_type=jax.ShapeDtypeStruct((_N, _C), jnp.float32),
        scratch_types=[
            pltpu.VMEM((_RPW // _G, _G), jnp.int32),
            pltpu.VMEM((2, _R, _CP), jnp.float32),
            pltpu.VMEM((_R, _C), jnp.float32),
            pltpu.SemaphoreType.DMA,
            pltpu.SemaphoreType.DMA,
        ],
    )
    def gather_kernel(idx_hbm, table_hbm, out_hbm, idx_v, rows_v, pack_v, sem0, sem1):
        sems = (sem0, sem1)
        wid = lax.axis_index("s") * _NC + lax.axis_index("c")
        base = wid * _RPW
        off = (base // _HW) * _K

        pltpu.sync_copy(idx_hbm.at[pl.ds(wid * (_RPW // _G), _RPW // _G)], idx_v)

        def add_off(r, carry):
            for i in range(_G // 16):
                sl = pl.ds(i * 16, 16)
                idx_v[r, sl] = idx_v[r, sl] + off
            return carry

        lax.fori_loop(0, _RPW // _G, add_off, 0)

        def fire(g, b):
            for j in range(_GPC):
                pltpu.async_copy(
                    table_hbm.at[idx_v.at[g * _GPC + j]],
                    rows_v.at[b].at[pl.ds(j * _G, _G)],
                    sems[b],
                )

        def drain(b):
            pltpu.make_async_copy(
                table_hbm.at[pl.ds(0, _R)], rows_v.at[b], sems[b]
            ).wait()

        def compact(b):
            def rt(r8, carry):
                for rr in range(8):
                    r = r8 * 8 + rr
                    for i in range(_C // 16):
                        sl = pl.ds(i * 16, 16)
                        pack_v[r, sl] = rows_v[b, r, sl]
                return carry

            lax.fori_loop(0, _R // 8, rt, 0)

        def writeback(g):
            pltpu.sync_copy(pack_v, out_hbm.at[pl.ds(base + g * _R, _R)])

        fire(0, 0)
        fire(1, 1)

        def body(h, carry):
            g = 2 * h
            more = h + 1 < _NCHUNK // 2
            drain(0)
            compact(0)

            @pl.when(more)
            def _():
                fire(g + 2, 0)

            writeback(g)
            drain(1)
            compact(1)

            @pl.when(more)
            def _():
                fire(g + 3, 1)

            writeback(g + 1)
            return carry

        lax.fori_loop(0, _NCHUNK // 2, body, 0)

    return gather_kernel


_gather = jax.jit(_build())


def kernel(pooled_feature_map, superpixel_map):
    table = jnp.pad(pooled_feature_map, ((0, 0), (0, 0), (0, _CP - _C)))
    table = table.reshape(_B * _K, _CP)
    idx = superpixel_map.reshape(_N // _G, _G)
    out = _gather(idx, table)
    return out.reshape(_B, _H, _W, _C)
